# Initial kernel scaffold; baseline (speedup 1.0000x reference)
import jax
import jax.numpy as jnp
from jax.experimental import pallas as pl


def kernel(actor_emb, topic_ids, Wa, ba, table, Wt, bt, scale):
    raise NotImplementedError("write your pallas kernel here")


# speedup vs baseline: 2.4598x; 2.4598x over previous
import functools

import jax
import jax.numpy as jnp
from jax import lax
from jax.experimental import pallas as pl
from jax.experimental.pallas import tpu as pltpu
from jax.experimental.pallas import tpu_sc as plsc

D = 128
B = 4096
L = 50
N_TOK = B * L

NC = 2
NS = 16
NW = NC * NS
CHUNK = 128
B_PER_W = N_TOK // NW
CHUNKS_PER_W = B_PER_W // CHUNK
CHUNKS_PAD = 56

def _sc_gather_body(table_hbm, idx_hbm, out_hbm, idx_v, rows_v, sem):
    wid = lax.axis_index("s") * NC + lax.axis_index("c")
    row0 = wid * CHUNKS_PER_W

    pltpu.sync_copy(idx_hbm.at[pl.ds(wid * CHUNKS_PAD, CHUNKS_PAD)], idx_v)

    def body(j, carry):
        pltpu.async_copy(table_hbm.at[idx_v.at[j]], rows_v, sem).wait()
        pltpu.sync_copy(rows_v, out_hbm.at[pl.ds((row0 + j) * CHUNK, CHUNK)])
        return carry

    lax.fori_loop(0, CHUNKS_PER_W, body, 0)


@functools.cache
def _sc_gather():
    mesh = plsc.VectorSubcoreMesh(core_axis_name="c", subcore_axis_name="s",
                                  num_cores=NC, num_subcores=NS)
    return pl.kernel(
        _sc_gather_body,
        out_type=jax.ShapeDtypeStruct((N_TOK, D), jnp.float32),
        mesh=mesh,
        scratch_types=[
            pltpu.VMEM((CHUNKS_PAD, CHUNK), jnp.int32),
            pltpu.VMEM((CHUNK, D), jnp.float32),
            pltpu.SemaphoreType.DMA,
        ],
    )

TC_BLOCK = 2048
N_BLOCKS = N_TOK // TC_BLOCK


def _tc_body(x_ref, g_ref, wa_ref, ba_ref, wt_ref, bt_ref, scale_ref,
             out_ref, m_ref, uv_ref):
    @pl.when(pl.program_id(0) == 0)
    def _():
        wa = wa_ref[...]
        wt = wt_ref[...]
        m_ref[...] = lax.dot_general(
            wa, wt, (((0,), (0,)), ((), ())),
            preferred_element_type=jnp.float32)
        uv_ref[0:1, :] = lax.dot_general(
            bt_ref[...], wa, (((1,), (0,)), ((), ())),
            preferred_element_type=jnp.float32)
        uv_ref[1:2, :] = lax.dot_general(
            ba_ref[...], wt, (((1,), (0,)), ((), ())),
            preferred_element_type=jnp.float32)

    x = x_ref[...]
    g = g_ref[...]
    z = lax.dot_general(x, m_ref[...], (((1,), (0,)), ((), ())),
                        preferred_element_type=jnp.float32)
    z = z + uv_ref[1:2, :]
    dots = jnp.sum(z * g, axis=1)
    xu = jnp.sum(x * uv_ref[0:1, :], axis=1)
    c = jnp.sum(ba_ref[...] * bt_ref[...])
    out_ref[...] = ((dots + xu + c) * scale_ref[0, 0]).reshape(TC_BLOCK // 128, 128)


def _tc_call(x, g, Wa, ba2, Wt, bt2, scale2):
    return pl.pallas_call(
        _tc_body,
        grid=(N_BLOCKS,),
        in_specs=[
            pl.BlockSpec((TC_BLOCK, D), lambda i: (i, 0)),
            pl.BlockSpec((TC_BLOCK, D), lambda i: (i, 0)),
            pl.BlockSpec((D, D), lambda i: (0, 0)),
            pl.BlockSpec((1, D), lambda i: (0, 0)),
            pl.BlockSpec((D, D), lambda i: (0, 0)),
            pl.BlockSpec((1, D), lambda i: (0, 0)),
            pl.BlockSpec(memory_space=pltpu.SMEM),
        ],
        out_specs=pl.BlockSpec((TC_BLOCK // 128, 128), lambda i: (i, 0)),
        out_shape=jax.ShapeDtypeStruct((N_TOK // 128, 128), jnp.float32),
        scratch_shapes=[
            pltpu.VMEM((D, D), jnp.float32),
            pltpu.VMEM((2, D), jnp.float32),
        ],
    )(x, g, Wa, ba2, Wt, bt2, scale2)


def kernel(actor_emb, topic_ids, Wa, ba, table, Wt, bt, scale):
    x = actor_emb.reshape(N_TOK, D)
    idx3d = topic_ids.reshape(NW, CHUNKS_PER_W, CHUNK).astype(jnp.int32)
    idx_pad = jnp.pad(idx3d, ((0, 0), (0, CHUNKS_PAD - CHUNKS_PER_W), (0, 0)))
    g = _sc_gather()(table, idx_pad.reshape(NW * CHUNKS_PAD, CHUNK))
    out2d = _tc_call(x, g, Wa, ba.reshape(1, D), Wt, bt.reshape(1, D),
                     scale.reshape(1, 1))
    return out2d.reshape(B, L)


# speedup vs baseline: 2.7998x; 1.1382x over previous
import functools

import jax
import jax.numpy as jnp
from jax import lax
from jax.experimental import pallas as pl
from jax.experimental.pallas import tpu as pltpu
from jax.experimental.pallas import tpu_sc as plsc

D = 128
B = 4096
L = 50
N_TOK = B * L

NC = 2
NS = 16
NW = NC * NS
CHUNK = 128
B_PER_W = N_TOK // NW
CHUNKS_PER_W = B_PER_W // CHUNK
CHUNKS_PAD = 56

def _sc_gather_body(table_hbm, idx_hbm, out_hbm, idx_v, rows_v, sem):
    wid = lax.axis_index("s") * NC + lax.axis_index("c")
    row0 = wid * CHUNKS_PER_W

    pltpu.sync_copy(idx_hbm.at[pl.ds(wid * CHUNKS_PAD, CHUNKS_PAD)], idx_v)

    def body(j, carry):
        pltpu.async_copy(table_hbm.at[idx_v.at[j]], rows_v, sem).wait()
        pltpu.sync_copy(rows_v, out_hbm.at[pl.ds((row0 + j) * CHUNK, CHUNK)])
        return carry

    lax.fori_loop(0, CHUNKS_PER_W, body, 0)


@functools.cache
def _sc_gather():
    mesh = plsc.VectorSubcoreMesh(core_axis_name="c", subcore_axis_name="s",
                                  num_cores=NC, num_subcores=NS)
    return pl.kernel(
        _sc_gather_body,
        out_type=jax.ShapeDtypeStruct((N_TOK, D), jnp.float32),
        mesh=mesh,
        scratch_types=[
            pltpu.VMEM((CHUNKS_PAD, CHUNK), jnp.int32),
            pltpu.VMEM((CHUNK, D), jnp.float32),
            pltpu.SemaphoreType.DMA,
        ],
    )

BA = 64
TC_BLOCK = BA * L
N_BLOCKS = B // BA


def _tc_body(x_ref, g_ref, wa_ref, ba_ref, wt_ref, bt_ref, scale_ref,
             out_ref, m_ref, uv_ref):
    @pl.when(pl.program_id(0) == 0)
    def _():
        wa = wa_ref[...]
        wt = wt_ref[...]
        m_ref[...] = lax.dot_general(
            wa, wt, (((0,), (0,)), ((), ())),
            preferred_element_type=jnp.float32)
        uv_ref[0:1, :] = lax.dot_general(
            bt_ref[...], wa, (((1,), (0,)), ((), ())),
            preferred_element_type=jnp.float32)
        uv_ref[1:2, :] = lax.dot_general(
            ba_ref[...], wt, (((1,), (0,)), ((), ())),
            preferred_element_type=jnp.float32)

    x = x_ref[...].reshape(TC_BLOCK, D)
    g = g_ref[...]
    z = lax.dot_general(x, m_ref[...], (((1,), (0,)), ((), ())),
                        preferred_element_type=jnp.float32)
    z = z + uv_ref[1:2, :]
    dots = jnp.sum(z * g, axis=1)
    xu = jnp.sum(x * uv_ref[0:1, :], axis=1)
    c = jnp.sum(ba_ref[...] * bt_ref[...])
    out_ref[...] = ((dots + xu + c) * scale_ref[0, 0]).reshape(BA, L)


def _tc_call(x, g, Wa, ba2, Wt, bt2, scale2):
    return pl.pallas_call(
        _tc_body,
        grid=(N_BLOCKS,),
        in_specs=[
            pl.BlockSpec((BA, L, D), lambda i: (i, 0, 0)),
            pl.BlockSpec((TC_BLOCK, D), lambda i: (i, 0)),
            pl.BlockSpec((D, D), lambda i: (0, 0)),
            pl.BlockSpec((1, D), lambda i: (0, 0)),
            pl.BlockSpec((D, D), lambda i: (0, 0)),
            pl.BlockSpec((1, D), lambda i: (0, 0)),
            pl.BlockSpec(memory_space=pltpu.SMEM),
        ],
        out_specs=pl.BlockSpec((BA, L), lambda i: (i, 0)),
        out_shape=jax.ShapeDtypeStruct((B, L), jnp.float32),
        scratch_shapes=[
            pltpu.VMEM((D, D), jnp.float32),
            pltpu.VMEM((2, D), jnp.float32),
        ],
    )(x, g, Wa, ba2, Wt, bt2, scale2)


def kernel(actor_emb, topic_ids, Wa, ba, table, Wt, bt, scale):
    idx3d = topic_ids.reshape(NW, CHUNKS_PER_W, CHUNK).astype(jnp.int32)
    idx_pad = jnp.pad(idx3d, ((0, 0), (0, CHUNKS_PAD - CHUNKS_PER_W), (0, 0)))
    g = _sc_gather()(table, idx_pad.reshape(NW * CHUNKS_PAD, CHUNK))
    return _tc_call(actor_emb, g, Wa, ba.reshape(1, D), Wt, bt.reshape(1, D),
                    scale.reshape(1, 1))


# speedup vs baseline: 2.9422x; 1.0509x over previous
import functools

import jax
import jax.numpy as jnp
from jax import lax
from jax.experimental import pallas as pl
from jax.experimental.pallas import tpu as pltpu
from jax.experimental.pallas import tpu_sc as plsc

D = 128
B = 4096
L = 50
N_TOK = B * L

NC = 2
NS = 16
NW = NC * NS
CHUNK = 128
B_PER_W = N_TOK // NW
CHUNKS_PER_W = B_PER_W // CHUNK
CHUNKS_PAD = 56

NBUF = 4


def _sc_gather_body(table_hbm, idx_hbm, out_hbm, idx_v, rows_v, gsem, ssem):
    wid = lax.axis_index("s") * NC + lax.axis_index("c")
    row0 = wid * CHUNKS_PER_W

    pltpu.sync_copy(idx_hbm.at[pl.ds(wid * CHUNKS_PAD, CHUNKS_PAD)], idx_v)

    def gather(j):
        return pltpu.async_copy(
            table_hbm.at[idx_v.at[j]], rows_v.at[j % NBUF], gsem)

    def gather_wait(j):
        pltpu.make_async_copy(
            table_hbm.at[idx_v.at[j]], rows_v.at[j % NBUF], gsem).wait()

    def store(j):
        return pltpu.async_copy(
            rows_v.at[j % NBUF], out_hbm.at[pl.ds((row0 + j) * CHUNK, CHUNK)],
            ssem)

    def store_wait(j):
        pltpu.make_async_copy(
            rows_v.at[j % NBUF], out_hbm.at[pl.ds((row0 + j) * CHUNK, CHUNK)],
            ssem).wait()

    for j in range(NBUF - 1):
        gather(j)

    def body(j, carry):
        gather_wait(j)
        store(j)

        @pl.when(j >= 1)
        def _():
            store_wait(j - 1)

        @pl.when(j + NBUF - 1 < CHUNKS_PER_W)
        def _():
            gather(j + NBUF - 1)

        return carry

    lax.fori_loop(0, CHUNKS_PER_W, body, 0)
    store_wait(CHUNKS_PER_W - 1)


@functools.cache
def _sc_gather():
    mesh = plsc.VectorSubcoreMesh(core_axis_name="c", subcore_axis_name="s",
                                  num_cores=NC, num_subcores=NS)
    return pl.kernel(
        _sc_gather_body,
        out_type=jax.ShapeDtypeStruct((N_TOK, D), jnp.float32),
        mesh=mesh,
        scratch_types=[
            pltpu.VMEM((CHUNKS_PAD, CHUNK), jnp.int32),
            pltpu.VMEM((NBUF, CHUNK, D), jnp.float32),
            pltpu.SemaphoreType.DMA,
            pltpu.SemaphoreType.DMA,
        ],
    )

BA = 64
TC_BLOCK = BA * L
N_BLOCKS = B // BA


def _tc_body(x_ref, g_ref, wa_ref, ba_ref, wt_ref, bt_ref, scale_ref,
             out_ref, m_ref, uv_ref):
    @pl.when(pl.program_id(0) == 0)
    def _():
        wa = wa_ref[...]
        wt = wt_ref[...]
        m_ref[...] = lax.dot_general(
            wa, wt, (((0,), (0,)), ((), ())),
            preferred_element_type=jnp.float32)
        uv_ref[0:1, :] = lax.dot_general(
            bt_ref[...], wa, (((1,), (0,)), ((), ())),
            preferred_element_type=jnp.float32)
        uv_ref[1:2, :] = lax.dot_general(
            ba_ref[...], wt, (((1,), (0,)), ((), ())),
            preferred_element_type=jnp.float32)

    x = x_ref[...].reshape(TC_BLOCK, D)
    g = g_ref[...]
    z = lax.dot_general(x, m_ref[...], (((1,), (0,)), ((), ())),
                        preferred_element_type=jnp.float32)
    z = z + uv_ref[1:2, :]
    dots = jnp.sum(z * g, axis=1)
    xu = jnp.sum(x * uv_ref[0:1, :], axis=1)
    c = jnp.sum(ba_ref[...] * bt_ref[...])
    out_ref[...] = ((dots + xu + c) * scale_ref[0, 0]).reshape(BA, L)


def _tc_call(x, g, Wa, ba2, Wt, bt2, scale2):
    return pl.pallas_call(
        _tc_body,
        grid=(N_BLOCKS,),
        in_specs=[
            pl.BlockSpec((BA, L, D), lambda i: (i, 0, 0)),
            pl.BlockSpec((TC_BLOCK, D), lambda i: (i, 0)),
            pl.BlockSpec((D, D), lambda i: (0, 0)),
            pl.BlockSpec((1, D), lambda i: (0, 0)),
            pl.BlockSpec((D, D), lambda i: (0, 0)),
            pl.BlockSpec((1, D), lambda i: (0, 0)),
            pl.BlockSpec(memory_space=pltpu.SMEM),
        ],
        out_specs=pl.BlockSpec((BA, L), lambda i: (i, 0)),
        out_shape=jax.ShapeDtypeStruct((B, L), jnp.float32),
        scratch_shapes=[
            pltpu.VMEM((D, D), jnp.float32),
            pltpu.VMEM((2, D), jnp.float32),
        ],
    )(x, g, Wa, ba2, Wt, bt2, scale2)


def kernel(actor_emb, topic_ids, Wa, ba, table, Wt, bt, scale):
    idx3d = topic_ids.reshape(NW, CHUNKS_PER_W, CHUNK).astype(jnp.int32)
    idx_pad = jnp.pad(idx3d, ((0, 0), (0, CHUNKS_PAD - CHUNKS_PER_W), (0, 0)))
    g = _sc_gather()(table, idx_pad.reshape(NW * CHUNKS_PAD, CHUNK))
    return _tc_call(actor_emb, g, Wa, ba.reshape(1, D), Wt, bt.reshape(1, D),
                    scale.reshape(1, 1))
